# initial kernel scaffold (unmeasured)
import functools

import jax
import jax.numpy as jnp
from jax import lax
from jax.experimental import pallas as pl
from jax.experimental.pallas import tpu as pltpu

N_DEV = 16
SCALE = 64 ** -0.5
NEG_INF = -1e30


def _attn_step(q, k, v, o, m, l):
    s = lax.dot_general(
        q, k, (((2,), (2,)), ((0,), (0,))), preferred_element_type=jnp.float32
    ) * SCALE
    m_new = jnp.maximum(m, jnp.max(s, axis=-1, keepdims=True))
    alpha = jnp.exp(m - m_new)
    p = jnp.exp(s - m_new)
    l_new = l * alpha + jnp.sum(p, axis=-1, keepdims=True)
    pv = lax.dot_general(
        p, v, (((2,), (1,)), ((0,), (0,))), preferred_element_type=jnp.float32
    )
    o_new = o * alpha + pv
    return o_new, m_new, l_new


def _body(q_ref, k_ref, v_ref, out_ref, kv_comm, send_sems, recv_sems):
    my = lax.axis_index("i")
    right = lax.rem(my + 1, N_DEV)
    left = lax.rem(my + N_DEV - 1, N_DEV)

    barrier_sem = pltpu.get_barrier_semaphore()
    for nbr in (left, right):
        pl.semaphore_signal(
            barrier_sem, inc=1, device_id=(nbr,),
            device_id_type=pl.DeviceIdType.MESH,
        )
    pl.semaphore_wait(barrier_sem, 2)

    kv_comm[0, 0] = k_ref[...]
    kv_comm[0, 1] = v_ref[...]

    q = q_ref[...]
    bh, qs, d = q.shape
    m = jnp.full((bh, qs, 1), NEG_INF, jnp.float32)
    l = jnp.zeros((bh, qs, 1), jnp.float32)
    o = jnp.zeros((bh, qs, d), jnp.float32)

    o, m, l = _attn_step(q, k_ref[...], v_ref[...], o, m, l)

    for h in range(N_DEV - 1):
        rdma = pltpu.make_async_remote_copy(
            src_ref=kv_comm.at[h],
            dst_ref=kv_comm.at[h + 1],
            send_sem=send_sems.at[h],
            recv_sem=recv_sems.at[h],
            device_id=(right,),
            device_id_type=pl.DeviceIdType.MESH,
        )
        rdma.start()
        rdma.wait()
        o, m, l = _attn_step(q, kv_comm[h + 1, 0], kv_comm[h + 1, 1], o, m, l)

    out_ref[...] = o / l

    @functools.partial(pl.run_scoped, sem=pltpu.SemaphoreType.REGULAR)
    def _(sem):
        for nbr in (left, right):
            pl.semaphore_signal(
                sem, inc=1, device_id=(nbr,),
                device_id_type=pl.DeviceIdType.MESH,
            )
        pl.semaphore_wait(sem, 2)


def kernel(Q, K, V):
    b, s, h, d = Q.shape
    bh = b * h
    qt = Q.transpose(0, 2, 1, 3).reshape(bh, s, d)
    kt = K.transpose(0, 2, 1, 3).reshape(bh, s, d)
    vt = V.transpose(0, 2, 1, 3).reshape(bh, s, d)

    out = pl.pallas_call(
        _body,
        out_shape=jax.ShapeDtypeStruct((bh, s, d), jnp.float32),
        in_specs=[pl.BlockSpec(memory_space=pltpu.VMEM)] * 3,
        out_specs=pl.BlockSpec(memory_space=pltpu.VMEM),
        scratch_shapes=[
            pltpu.VMEM((N_DEV, 2, bh, s, d), jnp.float32),
            pltpu.SemaphoreType.DMA((N_DEV - 1,)),
            pltpu.SemaphoreType.DMA((N_DEV - 1,)),
        ],
        compiler_params=pltpu.CompilerParams(collective_id=0),
    )(qt, kt, vt)
    return out.reshape(b, h, s, d).transpose(0, 2, 1, 3)


# baseline (device time: 379445 ns/iter reference)
import functools

import jax
import jax.numpy as jnp
from jax import lax
from jax.experimental import pallas as pl
from jax.experimental.pallas import tpu as pltpu

N_DEV = 16
SCALE = 64 ** -0.5
NEG_INF = -1e30


def _body(q_ref, k_ref, v_ref, out_ref, kv_comm, ot_acc, m_acc, l_acc,
          send_sems, recv_sems):
    my = lax.axis_index("i")
    right = lax.rem(my + 1, N_DEV)
    left = lax.rem(my + N_DEV - 1, N_DEV)

    barrier_sem = pltpu.get_barrier_semaphore()
    for nbr in (left, right):
        pl.semaphore_signal(
            barrier_sem, inc=1, device_id=(nbr,),
            device_id_type=pl.DeviceIdType.MESH,
        )
    pl.semaphore_wait(barrier_sem, 2)

    kv_comm[0, 0] = k_ref[...]
    kv_comm[0, 1] = v_ref[...]

    m_acc[...] = jnp.full(m_acc.shape, NEG_INF, jnp.float32)
    l_acc[...] = jnp.zeros(l_acc.shape, jnp.float32)
    ot_acc[...] = jnp.zeros(ot_acc.shape, jnp.float32)

    def attn_step(slot):
        qt = q_ref[...]
        kt = kv_comm[slot, 0]
        vt = kv_comm[slot, 1]
        m = m_acc[...]
        l = l_acc[...]
        s = lax.dot_general(
            qt, kt, (((1,), (1,)), ((0,), (0,))),
            preferred_element_type=jnp.float32,
        ) * SCALE
        m_new = jnp.maximum(m, jnp.max(s, axis=-1))
        alpha = jnp.exp(m - m_new)
        p = jnp.exp(s - m_new[:, :, None])
        l_new = l * alpha + jnp.sum(p, axis=-1)
        pv = lax.dot_general(
            vt, p, (((2,), (2,)), ((0,), (0,))),
            preferred_element_type=jnp.float32,
        )
        ot_acc[...] = ot_acc[...] * alpha[:, None, :] + pv
        m_acc[...] = m_new
        l_acc[...] = l_new

    def hop(h, _):
        rdma = pltpu.make_async_remote_copy(
            src_ref=kv_comm.at[h],
            dst_ref=kv_comm.at[h + 1],
            send_sem=send_sems.at[h],
            recv_sem=recv_sems.at[h],
            device_id=(right,),
            device_id_type=pl.DeviceIdType.MESH,
        )
        rdma.start()
        attn_step(h)
        rdma.wait()
        return _

    lax.fori_loop(0, N_DEV - 1, hop, None)
    attn_step(N_DEV - 1)

    out_ref[...] = ot_acc[...] / l_acc[...][:, None, :]

    @functools.partial(pl.run_scoped, sem=pltpu.SemaphoreType.REGULAR)
    def _(sem):
        for nbr in (left, right):
            pl.semaphore_signal(
                sem, inc=1, device_id=(nbr,),
                device_id_type=pl.DeviceIdType.MESH,
            )
        pl.semaphore_wait(sem, 2)


def kernel(Q, K, V):
    b, s, h, d = Q.shape
    bh = b * h
    qt = Q.transpose(0, 2, 3, 1).reshape(bh, d, s)
    kt = K.transpose(0, 2, 3, 1).reshape(bh, d, s)
    vt = V.transpose(0, 2, 3, 1).reshape(bh, d, s)

    out = pl.pallas_call(
        _body,
        out_shape=jax.ShapeDtypeStruct((bh, d, s), jnp.float32),
        in_specs=[pl.BlockSpec(memory_space=pltpu.VMEM)] * 3,
        out_specs=pl.BlockSpec(memory_space=pltpu.VMEM),
        scratch_shapes=[
            pltpu.VMEM((N_DEV, 2, bh, d, s), jnp.float32),
            pltpu.VMEM((bh, d, s), jnp.float32),
            pltpu.VMEM((bh, s), jnp.float32),
            pltpu.VMEM((bh, s), jnp.float32),
            pltpu.SemaphoreType.DMA((N_DEV - 1,)),
            pltpu.SemaphoreType.DMA((N_DEV - 1,)),
        ],
        compiler_params=pltpu.CompilerParams(
            collective_id=0, vmem_limit_bytes=60 * 1024 * 1024,
        ),
    )(qt, kt, vt)
    return out.reshape(b, h, d, s).transpose(0, 3, 1, 2)


# device time: 231060 ns/iter; 1.6422x vs baseline; 1.6422x over previous
import functools

import jax
import jax.numpy as jnp
from jax import lax
from jax.experimental import pallas as pl
from jax.experimental.pallas import tpu as pltpu

N_DEV = 16
NH = 8
SCALE = 64 ** -0.5
NEG_INF = -1e30


def _body(q_ref, k_ref, v_ref, out_ref, kvr_comm, kvl_comm,
          ot_acc, m_acc, l_acc,
          r_send_sems, r_recv_sems, l_send_sems, l_recv_sems):
    my = lax.axis_index("i")
    right = lax.rem(my + 1, N_DEV)
    left = lax.rem(my + N_DEV - 1, N_DEV)

    barrier_sem = pltpu.get_barrier_semaphore()
    for nbr in (left, right):
        pl.semaphore_signal(
            barrier_sem, inc=1, device_id=(nbr,),
            device_id_type=pl.DeviceIdType.MESH,
        )
    pl.semaphore_wait(barrier_sem, 2)

    kvr_comm[0, 0] = k_ref[0:NH]
    kvr_comm[0, 1] = v_ref[0:NH]
    kvl_comm[0, 0] = k_ref[NH:]
    kvl_comm[0, 1] = v_ref[NH:]

    m_acc[...] = jnp.full(m_acc.shape, NEG_INF, jnp.float32)
    l_acc[...] = jnp.zeros(l_acc.shape, jnp.float32)
    ot_acc[...] = jnp.zeros(ot_acc.shape, jnp.float32)

    def attn_step(slot):
        qt = q_ref[...]
        kt = jnp.concatenate([kvr_comm[slot, 0], kvl_comm[slot, 0]], axis=0)
        vt = jnp.concatenate([kvr_comm[slot, 1], kvl_comm[slot, 1]], axis=0)
        m = m_acc[...]
        l = l_acc[...]
        s = lax.dot_general(
            qt, kt, (((1,), (1,)), ((0,), (0,))),
            preferred_element_type=jnp.float32,
        ) * SCALE
        m_new = jnp.maximum(m, jnp.max(s, axis=-1))
        alpha = jnp.exp(m - m_new)
        p = jnp.exp(s - m_new[:, :, None])
        l_new = l * alpha + jnp.sum(p, axis=-1)
        pv = lax.dot_general(
            vt, p, (((2,), (2,)), ((0,), (0,))),
            preferred_element_type=jnp.float32,
        )
        ot_acc[...] = ot_acc[...] * alpha[:, None, :] + pv
        m_acc[...] = m_new
        l_acc[...] = l_new

    def _rdma(comm, send_sems, recv_sems, h, dev):
        return pltpu.make_async_remote_copy(
            src_ref=comm.at[h],
            dst_ref=comm.at[h + 1],
            send_sem=send_sems.at[h],
            recv_sem=recv_sems.at[h],
            device_id=(dev,),
            device_id_type=pl.DeviceIdType.MESH,
        )

    def hop(h, _):
        r = _rdma(kvr_comm, r_send_sems, r_recv_sems, h, right)
        lft = _rdma(kvl_comm, l_send_sems, l_recv_sems, h, left)
        r.start()
        lft.start()
        attn_step(h)
        r.wait_recv()
        lft.wait_recv()
        return _

    lax.fori_loop(0, N_DEV - 1, hop, None)
    attn_step(N_DEV - 1)

    out_ref[...] = ot_acc[...] / l_acc[...][:, None, :]

    def drain(h, _):
        _rdma(kvr_comm, r_send_sems, r_recv_sems, h, right).wait_send()
        _rdma(kvl_comm, l_send_sems, l_recv_sems, h, left).wait_send()
        return _

    lax.fori_loop(0, N_DEV - 1, drain, None)

    @functools.partial(pl.run_scoped, sem=pltpu.SemaphoreType.REGULAR)
    def _(sem):
        for nbr in (left, right):
            pl.semaphore_signal(
                sem, inc=1, device_id=(nbr,),
                device_id_type=pl.DeviceIdType.MESH,
            )
        pl.semaphore_wait(sem, 2)


def kernel(Q, K, V):
    b, s, h, d = Q.shape
    bh = b * h
    qt = Q.transpose(0, 2, 3, 1).reshape(bh, d, s)
    kt = K.transpose(0, 2, 3, 1).reshape(bh, d, s)
    vt = V.transpose(0, 2, 3, 1).reshape(bh, d, s)

    out = pl.pallas_call(
        _body,
        out_shape=jax.ShapeDtypeStruct((bh, d, s), jnp.float32),
        in_specs=[pl.BlockSpec(memory_space=pltpu.VMEM)] * 3,
        out_specs=pl.BlockSpec(memory_space=pltpu.VMEM),
        scratch_shapes=[
            pltpu.VMEM((N_DEV, 2, NH, d, s), jnp.float32),
            pltpu.VMEM((N_DEV, 2, NH, d, s), jnp.float32),
            pltpu.VMEM((bh, d, s), jnp.float32),
            pltpu.VMEM((bh, s), jnp.float32),
            pltpu.VMEM((bh, s), jnp.float32),
            pltpu.SemaphoreType.DMA((N_DEV - 1,)),
            pltpu.SemaphoreType.DMA((N_DEV - 1,)),
            pltpu.SemaphoreType.DMA((N_DEV - 1,)),
            pltpu.SemaphoreType.DMA((N_DEV - 1,)),
        ],
        compiler_params=pltpu.CompilerParams(
            collective_id=0, vmem_limit_bytes=60 * 1024 * 1024,
        ),
    )(qt, kt, vt)
    return out.reshape(b, h, d, s).transpose(0, 3, 1, 2)


# device time: 189188 ns/iter; 2.0057x vs baseline; 1.2213x over previous
import functools

import jax
import jax.numpy as jnp
from jax import lax
from jax.experimental import pallas as pl
from jax.experimental.pallas import tpu as pltpu

N_DEV = 16
NH = 8
NSUB = 2
SH = NH // NSUB
SCALE = 64 ** -0.5
NEG_INF = -1e30


def _body(q_ref, k_ref, v_ref, out_ref, kvr_comm, kvl_comm,
          ot_acc, m_acc, l_acc,
          r_send_sems, r_recv_sems, l_send_sems, l_recv_sems):
    my = lax.axis_index("i")
    right = lax.rem(my + 1, N_DEV)
    left = lax.rem(my + N_DEV - 1, N_DEV)

    barrier_sem = pltpu.get_barrier_semaphore()
    for nbr in (left, right):
        pl.semaphore_signal(
            barrier_sem, inc=1, device_id=(nbr,),
            device_id_type=pl.DeviceIdType.MESH,
        )
    pl.semaphore_wait(barrier_sem, 2)

    for j in range(NSUB):
        kvr_comm[0, j, 0] = k_ref[j * SH:(j + 1) * SH]
        kvr_comm[0, j, 1] = v_ref[j * SH:(j + 1) * SH]
        kvl_comm[0, j, 0] = k_ref[NH + j * SH:NH + (j + 1) * SH]
        kvl_comm[0, j, 1] = v_ref[NH + j * SH:NH + (j + 1) * SH]

    m_acc[...] = jnp.full(m_acc.shape, NEG_INF, jnp.float32)
    l_acc[...] = jnp.zeros(l_acc.shape, jnp.float32)
    ot_acc[...] = jnp.zeros(ot_acc.shape, jnp.float32)

    chains = [
        (kvr_comm, r_send_sems, r_recv_sems, right, 0),
        (kvl_comm, l_send_sems, l_recv_sems, left, NH),
    ]

    def attn_step(rows, slot, j, comm):
        qt = q_ref[rows]
        kt = comm[slot, j, 0]
        vt = comm[slot, j, 1]
        m = m_acc[rows]
        l = l_acc[rows]
        s = lax.dot_general(
            qt, kt, (((1,), (1,)), ((0,), (0,))),
            preferred_element_type=jnp.float32,
        ) * SCALE
        m_new = jnp.maximum(m, jnp.max(s, axis=-1))
        alpha = jnp.exp(m - m_new)
        p = jnp.exp(s - m_new[:, :, None])
        l_new = l * alpha + jnp.sum(p, axis=-1)
        pv = lax.dot_general(
            vt, p, (((2,), (2,)), ((0,), (0,))),
            preferred_element_type=jnp.float32,
        )
        ot_acc[rows] = ot_acc[rows] * alpha[:, None, :] + pv
        m_acc[rows] = m_new
        l_acc[rows] = l_new

    def compute_slot(slot):
        for comm, _, _, _, base in chains:
            for j in range(NSUB):
                attn_step(slice(base + j * SH, base + (j + 1) * SH),
                          slot, j, comm)

    def _rdma(comm, send_sems, recv_sems, h, j, dev):
        return pltpu.make_async_remote_copy(
            src_ref=comm.at[h, j],
            dst_ref=comm.at[h + 1, j],
            send_sem=send_sems.at[h, j],
            recv_sem=recv_sems.at[h, j],
            device_id=(dev,),
            device_id_type=pl.DeviceIdType.MESH,
        )

    def start_hop(h):
        for comm, ss, rs, dev, _ in chains:
            for j in range(NSUB):
                _rdma(comm, ss, rs, h, j, dev).start()

    start_hop(0)
    compute_slot(0)

    def hop(h, carry):
        for j in range(NSUB):
            for comm, ss, rs, dev, _base in chains:
                _rdma(comm, ss, rs, h, j, dev).wait_recv()
                _rdma(comm, ss, rs, h + 1, j, dev).start()
        compute_slot(h + 1)
        return carry

    lax.fori_loop(0, N_DEV - 2, hop, None)

    for comm, ss, rs, dev, _ in chains:
        for j in range(NSUB):
            _rdma(comm, ss, rs, N_DEV - 2, j, dev).wait_recv()
    compute_slot(N_DEV - 1)

    out_ref[...] = ot_acc[...] / l_acc[...][:, None, :]

    def drain(h, carry):
        for comm, ss, rs, dev, _base in chains:
            for j in range(NSUB):
                _rdma(comm, ss, rs, h, j, dev).wait_send()
        return carry

    lax.fori_loop(0, N_DEV - 1, drain, None)

    @functools.partial(pl.run_scoped, sem=pltpu.SemaphoreType.REGULAR)
    def _(sem):
        for nbr in (left, right):
            pl.semaphore_signal(
                sem, inc=1, device_id=(nbr,),
                device_id_type=pl.DeviceIdType.MESH,
            )
        pl.semaphore_wait(sem, 2)


def kernel(Q, K, V):
    b, s, h, d = Q.shape
    bh = b * h
    qt = Q.transpose(0, 2, 3, 1).reshape(bh, d, s)
    kt = K.transpose(0, 2, 3, 1).reshape(bh, d, s)
    vt = V.transpose(0, 2, 3, 1).reshape(bh, d, s)

    out = pl.pallas_call(
        _body,
        out_shape=jax.ShapeDtypeStruct((bh, d, s), jnp.float32),
        in_specs=[pl.BlockSpec(memory_space=pltpu.VMEM)] * 3,
        out_specs=pl.BlockSpec(memory_space=pltpu.VMEM),
        scratch_shapes=[
            pltpu.VMEM((N_DEV, NSUB, 2, SH, d, s), jnp.float32),
            pltpu.VMEM((N_DEV, NSUB, 2, SH, d, s), jnp.float32),
            pltpu.VMEM((bh, d, s), jnp.float32),
            pltpu.VMEM((bh, s), jnp.float32),
            pltpu.VMEM((bh, s), jnp.float32),
            pltpu.SemaphoreType.DMA((N_DEV - 1, NSUB)),
            pltpu.SemaphoreType.DMA((N_DEV - 1, NSUB)),
            pltpu.SemaphoreType.DMA((N_DEV - 1, NSUB)),
            pltpu.SemaphoreType.DMA((N_DEV - 1, NSUB)),
        ],
        compiler_params=pltpu.CompilerParams(
            collective_id=0, vmem_limit_bytes=60 * 1024 * 1024,
        ),
    )(qt, kt, vt)
    return out.reshape(b, h, d, s).transpose(0, 3, 1, 2)
